# baseline (speedup 1.0000x reference)
import functools

import jax
import jax.numpy as jnp
from jax import lax
from jax.experimental import pallas as pl
from jax.experimental.pallas import tpu as pltpu
from jax.experimental.pallas import tpu_sc as plsc

N_NODES = 10000
FEAT = 128
NC = 2
NS = 16
NW = NC * NS
CHUNK = 128
CPW = 80
CPW0 = 96
CPW1 = 64
PCH = 16
EPAD = NW * CPW * CHUNK
NPAD = 10112
RPS = NPAD // NS
BLK = 632
LANES = 16

_mesh = plsc.VectorSubcoreMesh(core_axis_name="c", subcore_axis_name="s")


@functools.partial(
    pl.kernel,
    out_type=jax.ShapeDtypeStruct((NC * NPAD,), jnp.float32),
    mesh=_mesh,
    scratch_types=[
        pltpu.VMEM((CPW, CHUNK), jnp.int32),
        pltpu.VMEM((CHUNK,), jnp.float32),
        pltpu.VMEM((640,), jnp.float32),
        pltpu.VMEM_SHARED((NPAD,), jnp.float32),
        pltpu.SemaphoreType.DMA,
    ],
)
def _sc_degree(col_hbm, deg_hbm, colb_v, ones_v, zbuf_v, acc_sh, sem):
    cix = lax.axis_index("c")
    sid = lax.axis_index("s")
    wid = sid * NC + cix
    pltpu.sync_copy(col_hbm.at[wid], colb_v)
    for i in range(CHUNK // LANES):
        ones_v[pl.ds(i * LANES, LANES)] = jnp.ones((LANES,), jnp.float32)
    for i in range(640 // LANES):
        zbuf_v[pl.ds(i * LANES, LANES)] = jnp.zeros((LANES,), jnp.float32)

    @pl.when(sid < NS - 1)
    def _():
        pltpu.sync_copy(zbuf_v, acc_sh.at[pl.ds(sid * 640, 640)])

    @pl.when(sid == NS - 1)
    def _():
        pltpu.sync_copy(zbuf_v.at[pl.ds(0, 512)],
                        acc_sh.at[pl.ds((NS - 1) * 640, 512)])
    plsc.subcore_barrier()

    def fire(j, carry):
        pltpu.async_copy(ones_v, acc_sh.at[colb_v.at[j]], sem, add=True)
        return carry

    lax.fori_loop(0, CPW, fire, 0)

    def drain(j, carry):
        pltpu.make_async_copy(ones_v, acc_sh.at[colb_v.at[0]], sem).wait()
        return carry

    lax.fori_loop(0, CPW, drain, 0)
    plsc.subcore_barrier()

    @pl.when(sid < NS - 1)
    def _():
        pltpu.sync_copy(
            acc_sh.at[pl.ds(sid * 640, 640)],
            deg_hbm.at[pl.ds(cix * NPAD + sid * 640, 640)],
        )

    @pl.when(sid == NS - 1)
    def _():
        pltpu.sync_copy(
            acc_sh.at[pl.ds((NS - 1) * 640, 512)],
            deg_hbm.at[pl.ds(cix * NPAD + (NS - 1) * 640, 512)],
        )


@functools.partial(
    pl.kernel,
    out_type=jax.ShapeDtypeStruct((NC * NPAD, FEAT), jnp.float32),
    mesh=_mesh,
    scratch_types=[
        pltpu.VMEM((PCH, CHUNK), jnp.int32),
        pltpu.VMEM((PCH, CHUNK), jnp.int32),
        pltpu.VMEM((CHUNK, FEAT), jnp.float32),
        pltpu.VMEM((CHUNK, FEAT), jnp.float32),
        pltpu.VMEM_SHARED((NPAD, FEAT), jnp.float32),
        pltpu.SemaphoreType.DMA,
        pltpu.SemaphoreType.DMA,
    ],
)
def _sc_edges(row_hbm, col_hbm, y_hbm, out_hbm,
              rowb_v, colb_v, buf_a, buf_b, acc_sh, sem_a, sem_b):
    cix = lax.axis_index("c")
    sid = lax.axis_index("s")

    zeros16 = jnp.zeros((LANES,), jnp.float32)

    def zrow(r, carry):
        for k in range(FEAT // LANES):
            buf_a[r, pl.ds(k * LANES, LANES)] = zeros16
        return carry

    lax.fori_loop(0, CHUNK, zrow, 0)
    for k in range(RPS // CHUNK):
        pltpu.sync_copy(buf_a, acc_sh.at[pl.ds(sid * RPS + k * CHUNK, CHUNK)])
    pltpu.sync_copy(
        buf_a.at[pl.ds(0, RPS - (RPS // CHUNK) * CHUNK)],
        acc_sh.at[pl.ds(sid * RPS + (RPS // CHUNK) * CHUNK,
                        RPS - (RPS // CHUNK) * CHUNK)],
    )
    plsc.subcore_barrier()

    def run_chunks(base, nphases):
        for p in range(nphases):
            pltpu.sync_copy(row_hbm.at[pl.ds(base + p * PCH, PCH)], rowb_v)
            pltpu.sync_copy(col_hbm.at[pl.ds(base + p * PCH, PCH)], colb_v)

            def body(j, carry):
                pltpu.async_copy(y_hbm.at[rowb_v.at[j]], buf_a, sem_a).wait()
                pltpu.sync_copy(buf_a, acc_sh.at[colb_v.at[j]], add=True)
                return carry

            lax.fori_loop(0, PCH, body, 0)

    @pl.when(cix == 0)
    def _():
        run_chunks(sid * CPW0, CPW0 // PCH)

    @pl.when(cix == 1)
    def _():
        run_chunks(NS * CPW0 + sid * CPW1, CPW1 // PCH)

    plsc.subcore_barrier()
    pltpu.sync_copy(
        acc_sh.at[pl.ds(sid * RPS, RPS)],
        out_hbm.at[pl.ds(cix * NPAD + sid * RPS, RPS)],
    )


def _tc_prep_body(deg_ref, x_ref, y_ref):
    deg = deg_ref[:, 0] + deg_ref[:, 1] + 1.0
    dis = lax.rsqrt(deg)
    y_ref[...] = x_ref[...] * dis[:, None]


def _tc_final_body(deg_ref, s_ref, y_ref, wm_ref, bm_ref, wl_ref, bl_ref,
                   mu_ref, ls_ref):
    deg = deg_ref[:, 0] + deg_ref[:, 1] + 1.0
    dis = lax.rsqrt(deg)
    agg = (s_ref[0] + s_ref[1] + y_ref[...]) * dis[:, None]
    mu_ref[...] = (
        jnp.dot(agg, wm_ref[...], preferred_element_type=jnp.float32,
                precision=lax.Precision.HIGHEST) + bm_ref[...]
    )
    ls_ref[...] = (
        jnp.dot(agg, wl_ref[...], preferred_element_type=jnp.float32,
                precision=lax.Precision.HIGHEST) + bl_ref[...]
    )


def kernel(x, edge_index, W_mu, b_mu, W_logstd, b_logstd):
    row = edge_index[0].astype(jnp.int32)
    col = edge_index[1].astype(jnp.int32)
    e = row.shape[0]
    row_p = jnp.concatenate(
        [row, jnp.zeros((EPAD - e,), jnp.int32)]).reshape(NW * CPW, CHUNK)
    col_p = jnp.concatenate(
        [col, jnp.full((EPAD - e,), N_NODES, jnp.int32)]).reshape(NW * CPW, CHUNK)
    x_p = jnp.concatenate(
        [x, jnp.zeros((NPAD - N_NODES, FEAT), jnp.float32)])

    deg = _sc_degree(col_p.reshape(NW, CPW, CHUNK)).reshape(NC, NPAD).T

    y = pl.pallas_call(
        _tc_prep_body,
        grid=(NPAD // BLK,),
        in_specs=[
            pl.BlockSpec((BLK, NC), lambda i: (i, 0)),
            pl.BlockSpec((BLK, FEAT), lambda i: (i, 0)),
        ],
        out_specs=pl.BlockSpec((BLK, FEAT), lambda i: (i, 0)),
        out_shape=jax.ShapeDtypeStruct((NPAD, FEAT), jnp.float32),
    )(deg, x_p)

    s = _sc_edges(row_p, col_p, y).reshape(NC, NPAD, FEAT)

    mu_p, ls_p = pl.pallas_call(
        _tc_final_body,
        grid=(NPAD // BLK,),
        in_specs=[
            pl.BlockSpec((BLK, NC), lambda i: (i, 0)),
            pl.BlockSpec((NC, BLK, FEAT), lambda i: (0, i, 0)),
            pl.BlockSpec((BLK, FEAT), lambda i: (i, 0)),
            pl.BlockSpec((FEAT, FEAT), lambda i: (0, 0)),
            pl.BlockSpec((1, FEAT), lambda i: (0, 0)),
            pl.BlockSpec((FEAT, FEAT), lambda i: (0, 0)),
            pl.BlockSpec((1, FEAT), lambda i: (0, 0)),
        ],
        out_specs=[
            pl.BlockSpec((BLK, FEAT), lambda i: (i, 0)),
            pl.BlockSpec((BLK, FEAT), lambda i: (i, 0)),
        ],
        out_shape=[
            jax.ShapeDtypeStruct((NPAD, FEAT), jnp.float32),
            jax.ShapeDtypeStruct((NPAD, FEAT), jnp.float32),
        ],
    )(deg, s, y, W_mu, b_mu.reshape(1, FEAT), W_logstd, b_logstd.reshape(1, FEAT))

    return mu_p[:N_NODES], ls_p[:N_NODES]


# speedup vs baseline: 1.2098x; 1.2098x over previous
import functools

import jax
import jax.numpy as jnp
from jax import lax
from jax.experimental import pallas as pl
from jax.experimental.pallas import tpu as pltpu
from jax.experimental.pallas import tpu_sc as plsc

N_NODES = 10000
FEAT = 128
NC = 2
NS = 16
NW = NC * NS
CHUNK = 128
ECH = 64
ECW = 160
PCH = 32
NBUF = 4
CPW = 80
EPAD = NW * CPW * CHUNK
NPAD = 10112
RPS = NPAD // NS
BLK = 632
LANES = 16

_mesh = plsc.VectorSubcoreMesh(core_axis_name="c", subcore_axis_name="s")


@functools.partial(
    pl.kernel,
    out_type=jax.ShapeDtypeStruct((NC * NPAD,), jnp.float32),
    mesh=_mesh,
    scratch_types=[
        pltpu.VMEM((CPW, CHUNK), jnp.int32),
        pltpu.VMEM((CHUNK,), jnp.float32),
        pltpu.VMEM((640,), jnp.float32),
        pltpu.VMEM_SHARED((NPAD,), jnp.float32),
        pltpu.SemaphoreType.DMA,
    ],
)
def _sc_degree(col_hbm, deg_hbm, colb_v, ones_v, zbuf_v, acc_sh, sem):
    cix = lax.axis_index("c")
    sid = lax.axis_index("s")
    wid = sid * NC + cix
    pltpu.sync_copy(col_hbm.at[wid], colb_v)
    for i in range(CHUNK // LANES):
        ones_v[pl.ds(i * LANES, LANES)] = jnp.ones((LANES,), jnp.float32)
    for i in range(640 // LANES):
        zbuf_v[pl.ds(i * LANES, LANES)] = jnp.zeros((LANES,), jnp.float32)

    @pl.when(sid < NS - 1)
    def _():
        pltpu.sync_copy(zbuf_v, acc_sh.at[pl.ds(sid * 640, 640)])

    @pl.when(sid == NS - 1)
    def _():
        pltpu.sync_copy(zbuf_v.at[pl.ds(0, 512)],
                        acc_sh.at[pl.ds((NS - 1) * 640, 512)])
    plsc.subcore_barrier()

    def fire(j, carry):
        pltpu.async_copy(ones_v, acc_sh.at[colb_v.at[j]], sem, add=True)
        return carry

    lax.fori_loop(0, CPW, fire, 0)

    def drain(j, carry):
        pltpu.make_async_copy(ones_v, acc_sh.at[colb_v.at[0]], sem).wait()
        return carry

    lax.fori_loop(0, CPW, drain, 0)
    plsc.subcore_barrier()

    @pl.when(sid < NS - 1)
    def _():
        pltpu.sync_copy(
            acc_sh.at[pl.ds(sid * 640, 640)],
            deg_hbm.at[pl.ds(cix * NPAD + sid * 640, 640)],
        )

    @pl.when(sid == NS - 1)
    def _():
        pltpu.sync_copy(
            acc_sh.at[pl.ds((NS - 1) * 640, 512)],
            deg_hbm.at[pl.ds(cix * NPAD + (NS - 1) * 640, 512)],
        )


@functools.partial(
    pl.kernel,
    out_type=jax.ShapeDtypeStruct((NC * NPAD, FEAT), jnp.float32),
    mesh=_mesh,
    scratch_types=[
        pltpu.VMEM((PCH, ECH), jnp.int32),
        pltpu.VMEM((PCH, ECH), jnp.int32),
        pltpu.VMEM((ECH, FEAT), jnp.float32),
        pltpu.VMEM((ECH, FEAT), jnp.float32),
        pltpu.VMEM((ECH, FEAT), jnp.float32),
        pltpu.VMEM((ECH, FEAT), jnp.float32),
        pltpu.VMEM_SHARED((NPAD, FEAT), jnp.float32),
        pltpu.SemaphoreType.DMA,
        pltpu.SemaphoreType.DMA,
        pltpu.SemaphoreType.DMA,
        pltpu.SemaphoreType.DMA,
    ],
)
def _sc_edges(row_hbm, col_hbm, y_hbm, out_hbm,
              rowb_v, colb_v, buf_a, buf_b, buf_c, buf_d, acc_sh,
              sem_a, sem_b, sem_c, sem_d):
    cix = lax.axis_index("c")
    sid = lax.axis_index("s")
    bufs = [buf_a, buf_b, buf_c, buf_d]
    sems = [sem_a, sem_b, sem_c, sem_d]

    zeros16 = jnp.zeros((LANES,), jnp.float32)

    def zrow(r, carry):
        for k in range(FEAT // LANES):
            buf_a[r, pl.ds(k * LANES, LANES)] = zeros16
        return carry

    lax.fori_loop(0, ECH, zrow, 0)
    for k in range(RPS // ECH):
        pltpu.sync_copy(buf_a, acc_sh.at[pl.ds(sid * RPS + k * ECH, ECH)])
    pltpu.sync_copy(
        buf_a.at[pl.ds(0, RPS - (RPS // ECH) * ECH)],
        acc_sh.at[pl.ds(sid * RPS + (RPS // ECH) * ECH,
                        RPS - (RPS // ECH) * ECH)],
    )
    plsc.subcore_barrier()

    base = (sid * NC + cix) * ECW
    for p in range(ECW // PCH):
        pltpu.sync_copy(row_hbm.at[pl.ds(base + p * PCH, PCH)], rowb_v)
        pltpu.sync_copy(col_hbm.at[pl.ds(base + p * PCH, PCH)], colb_v)
        for b in range(NBUF):
            pltpu.async_copy(y_hbm.at[rowb_v.at[b]], bufs[b], sems[b])

        def body(q, carry):
            for b in range(NBUF):
                j = NBUF * q + b
                pltpu.make_async_copy(y_hbm.at[rowb_v.at[0]],
                                      bufs[b], sems[b]).wait()
                pltpu.sync_copy(bufs[b], acc_sh.at[colb_v.at[j]], add=True)

                @pl.when(j + NBUF < PCH)
                def _():
                    pltpu.async_copy(y_hbm.at[rowb_v.at[j + NBUF]],
                                     bufs[b], sems[b])
            return carry

        lax.fori_loop(0, PCH // NBUF, body, 0)

    plsc.subcore_barrier()
    pltpu.sync_copy(
        acc_sh.at[pl.ds(sid * RPS, RPS)],
        out_hbm.at[pl.ds(cix * NPAD + sid * RPS, RPS)],
    )


def _tc_prep_body(deg_ref, x_ref, y_ref):
    deg = deg_ref[:, 0] + deg_ref[:, 1] + 1.0
    dis = lax.rsqrt(deg)
    y_ref[...] = x_ref[...] * dis[:, None]


def _tc_final_body(deg_ref, s_ref, y_ref, wm_ref, bm_ref, wl_ref, bl_ref,
                   mu_ref, ls_ref):
    deg = deg_ref[:, 0] + deg_ref[:, 1] + 1.0
    dis = lax.rsqrt(deg)
    agg = (s_ref[0] + s_ref[1] + y_ref[...]) * dis[:, None]
    mu_ref[...] = (
        jnp.dot(agg, wm_ref[...], preferred_element_type=jnp.float32,
                precision=lax.Precision.HIGHEST) + bm_ref[...]
    )
    ls_ref[...] = (
        jnp.dot(agg, wl_ref[...], preferred_element_type=jnp.float32,
                precision=lax.Precision.HIGHEST) + bl_ref[...]
    )


def kernel(x, edge_index, W_mu, b_mu, W_logstd, b_logstd):
    row = edge_index[0].astype(jnp.int32)
    col = edge_index[1].astype(jnp.int32)
    e = row.shape[0]
    row_p = jnp.concatenate(
        [row, jnp.zeros((EPAD - e,), jnp.int32)]).reshape(NW * ECW, ECH)
    col_p = jnp.concatenate(
        [col, jnp.full((EPAD - e,), N_NODES, jnp.int32)]).reshape(NW * ECW, ECH)
    x_p = jnp.concatenate(
        [x, jnp.zeros((NPAD - N_NODES, FEAT), jnp.float32)])

    deg = _sc_degree(col_p.reshape(NW, CPW, CHUNK)).reshape(NC, NPAD).T


    y = pl.pallas_call(
        _tc_prep_body,
        grid=(NPAD // BLK,),
        in_specs=[
            pl.BlockSpec((BLK, NC), lambda i: (i, 0)),
            pl.BlockSpec((BLK, FEAT), lambda i: (i, 0)),
        ],
        out_specs=pl.BlockSpec((BLK, FEAT), lambda i: (i, 0)),
        out_shape=jax.ShapeDtypeStruct((NPAD, FEAT), jnp.float32),
    )(deg, x_p)

    s = _sc_edges(row_p, col_p, y).reshape(NC, NPAD, FEAT)

    mu_p, ls_p = pl.pallas_call(
        _tc_final_body,
        grid=(NPAD // BLK,),
        in_specs=[
            pl.BlockSpec((BLK, NC), lambda i: (i, 0)),
            pl.BlockSpec((NC, BLK, FEAT), lambda i: (0, i, 0)),
            pl.BlockSpec((BLK, FEAT), lambda i: (i, 0)),
            pl.BlockSpec((FEAT, FEAT), lambda i: (0, 0)),
            pl.BlockSpec((1, FEAT), lambda i: (0, 0)),
            pl.BlockSpec((FEAT, FEAT), lambda i: (0, 0)),
            pl.BlockSpec((1, FEAT), lambda i: (0, 0)),
        ],
        out_specs=[
            pl.BlockSpec((BLK, FEAT), lambda i: (i, 0)),
            pl.BlockSpec((BLK, FEAT), lambda i: (i, 0)),
        ],
        out_shape=[
            jax.ShapeDtypeStruct((NPAD, FEAT), jnp.float32),
            jax.ShapeDtypeStruct((NPAD, FEAT), jnp.float32),
        ],
    )(deg, s, y, W_mu, b_mu.reshape(1, FEAT), W_logstd, b_logstd.reshape(1, FEAT))

    return mu_p[:N_NODES], ls_p[:N_NODES]
